# baseline (speedup 1.0000x reference)
import functools

import numpy as np
import jax
import jax.numpy as jnp
from jax import lax
from jax.experimental import pallas as pl
from jax.experimental.pallas import tpu as pltpu


def _band_masks(Wd):
    m = np.zeros((3, Wd, Wd), np.float32)
    for j in range(Wd):
        for dx in range(3):
            i = j + dx - 1
            if 0 <= i < Wd:
                m[dx, j, i] = 1.0
    return m


def _up_masks(Wu):
    Wd = 2 * Wu
    m = np.zeros((2, Wd, Wu), np.float32)
    for w in range(Wu):
        for dj in range(2):
            m[dj, 2 * w + dj, w] = 1.0
    return m


def _mod_sel(C, W):
    s = np.zeros((W * C, C), np.float32)
    for r in range(W * C):
        s[r, r % C] = 1.0
    return s


def _interleave_mats(H):
    s = np.zeros((2, H // 2, H), np.float32)
    for i in range(H // 2):
        s[0, i, 2 * i] = 1.0
        s[1, i, 2 * i + 1] = 1.0
    return s


def _interleave_tap_mats(H):
    s = _interleave_mats(H)
    shm1 = np.eye(H, k=1, dtype=np.float32)
    shp1 = np.eye(H, k=-1, dtype=np.float32)
    return np.stack([s[0], s[1], s[0] @ shm1, s[1] @ shm1,
                     s[0] @ shp1, s[1] @ shp1])


def _shift_mats(H):
    return np.stack([np.eye(H, k=1, dtype=np.float32),
                     np.eye(H, k=-1, dtype=np.float32)])


def _builder_kernel(Cout,
                    w1p_ref, w2p_ref, wtp_ref, ro_ref, rcT_ref, ruT_ref,
                    mb_ref, mup_ref, w1u_ref, w1f_ref, w2b_ref, mu_ref):
    bf16 = jnp.bfloat16
    f32 = jnp.float32
    ro = ro_ref[...]
    rcT = rcT_ref[...]
    ruT = ruT_ref[...]

    def big(wsmall, rT):
        t = jnp.dot(ro, wsmall, preferred_element_type=f32)
        return jnp.dot(t, rT, preferred_element_type=f32)

    for k in range(3):
        au = af = az = None
        for dx in range(3):
            wkx = w1p_ref[3 * k + dx]
            m = mb_ref[dx]
            tu = big(wkx[:, :Cout], rcT) * m
            tf = big(wkx[:, Cout:], rcT) * m
            t2 = big(w2p_ref[3 * k + dx], rcT) * m
            au = tu if au is None else au + tu
            af = tf if af is None else af + tf
            az = t2 if az is None else az + t2
        w1u_ref[k] = au.astype(bf16)
        w1f_ref[k] = af.astype(bf16)
        w2b_ref[k] = az.astype(bf16)

    for p in range(2):
        acc = None
        for dj in range(2):
            t = big(wtp_ref[2 * p + dj], ruT) * mup_ref[dj]
            acc = t if acc is None else acc + t
        mu_ref[p] = acc.astype(bf16)


def _build_weight_mats(w1, w2, wt, Wu):
    Cout = w2.shape[0]
    Cin = wt.shape[0]
    Wd = 2 * Wu
    Nw = Cout * Wd
    Ku = Cin * Wu
    f32 = jnp.float32
    w1p = jnp.transpose(w1, (2, 3, 0, 1)).reshape(9, Cout, 2 * Cout)
    w2p = jnp.transpose(w2, (2, 3, 0, 1)).reshape(9, Cout, Cout)
    wtp = jnp.transpose(wt, (2, 3, 1, 0)).reshape(4, Cout, Cin)
    ro = _mod_sel(Cout, Wd)
    rcT = _mod_sel(Cout, Wd).T
    ruT = _mod_sel(Cin, Wu).T
    ones_oc = np.ones((Cout, Cout), np.float32)
    ones_ocu = np.ones((Cout, Cin), np.float32)
    mb = np.stack([np.kron(m, ones_oc) for m in _band_masks(Wd)])
    mup = np.stack([np.kron(m, ones_ocu) for m in _up_masks(Wu)])

    return pl.pallas_call(
        functools.partial(_builder_kernel, Cout),
        out_shape=(
            jax.ShapeDtypeStruct((3, Nw, Nw), jnp.bfloat16),
            jax.ShapeDtypeStruct((3, Nw, Nw), jnp.bfloat16),
            jax.ShapeDtypeStruct((3, Nw, Nw), jnp.bfloat16),
            jax.ShapeDtypeStruct((2, Nw, Ku), jnp.bfloat16),
        ),
    )(w1p.astype(f32), w2p.astype(f32), wtp.astype(f32),
      jnp.asarray(ro), jnp.asarray(rcT), jnp.asarray(ruT),
      jnp.asarray(mb), jnp.asarray(mup))


def _dot(a, b):
    return jnp.dot(a, b, preferred_element_type=jnp.float32)


def _fused_kernel(Hu, Wu, Cin, Cout,
                  fu_ref, fd_ref, mu_ref, w1u_ref, w1f_ref, w2_ref,
                  si_ref, sh_ref, eyu_ref, eyd_ref, eyf_ref,
                  btb_ref, b1b_ref, b2b_ref, o_ref):
    bf16 = jnp.bfloat16
    f32 = jnp.float32
    Wd = 2 * Wu
    Hd = 2 * Hu
    Nw = Cout * Wd

    xfu = lax.dot_general(eyu_ref[...], fu_ref[0], (((1,), (2,)), ((), ())),
                          preferred_element_type=f32)
    xfu = xfu.reshape(Cin * Wu, Hu).astype(bf16)
    fdp = lax.dot_general(eyd_ref[...], fd_ref[0], (((1,), (2,)), ((), ())),
                          preferred_element_type=f32)
    fdp = fdp.reshape(Nw, Hd).astype(bf16)

    up_e = _dot(mu_ref[0], xfu).astype(bf16)
    up_o = _dot(mu_ref[1], xfu).astype(bf16)
    up = (_dot(up_e, si_ref[0]) + _dot(up_o, si_ref[1])
          + btb_ref[0]).astype(bf16)
    upm = (_dot(up_e, si_ref[2]) + _dot(up_o, si_ref[3])
           + btb_ref[1]).astype(bf16)
    upp = (_dot(up_e, si_ref[4]) + _dot(up_o, si_ref[5])
           + btb_ref[2]).astype(bf16)
    fdm = _dot(fdp, sh_ref[0]).astype(bf16)
    fdpp = _dot(fdp, sh_ref[1]).astype(bf16)

    h1 = (_dot(w1u_ref[0], upm) + _dot(w1f_ref[0], fdm)
          + _dot(w1u_ref[1], up) + _dot(w1f_ref[1], fdp)
          + _dot(w1u_ref[2], upp) + _dot(w1f_ref[2], fdpp))
    h1 = jnp.maximum(h1 + b1b_ref[...], 0.0).astype(bf16)

    h1m = _dot(h1, sh_ref[0]).astype(bf16)
    h1p = _dot(h1, sh_ref[1]).astype(bf16)
    h2 = (_dot(w2_ref[0], h1m) + _dot(w2_ref[1], h1)
          + _dot(w2_ref[2], h1p))
    h2 = jnp.maximum(h2 + b2b_ref[...], 0.0)

    o3 = lax.dot_general(h2.reshape(Wd, Cout, Hd), eyf_ref[...],
                         (((0,), (0,)), ((), ())), preferred_element_type=f32)
    o_ref[0] = o3.astype(bf16)


def kernel(from_down, from_up, wt, bt, w1, b1, w2, b2):
    N, Cout, Hd, Wd = from_down.shape
    _, Cin, Hu, Wu = from_up.shape
    bf16 = jnp.bfloat16
    Ku = Cin * Wu
    Nw = Cout * Wd

    w1u, w1f, w2b, mu = _build_weight_mats(w1, w2, wt, Wu)
    si = jnp.asarray(_interleave_tap_mats(Hd), dtype=bf16)
    sh = jnp.asarray(_shift_mats(Hd), dtype=bf16)
    eyu = jnp.asarray(np.eye(Wu, dtype=np.float32), dtype=bf16)
    eyd = jnp.asarray(np.eye(Wd, dtype=np.float32), dtype=bf16)
    eyf = jnp.asarray(np.eye(Wd, dtype=np.float32))
    colmask = np.ones((3, 1, Hd), np.float32)
    colmask[1, 0, 0] = 0.0
    colmask[2, 0, Hd - 1] = 0.0
    btb = (jnp.tile(bt.astype(jnp.float32), Wd)[None, :, None]
           * jnp.asarray(colmask))
    b1b = jnp.broadcast_to(
        jnp.tile(b1.astype(jnp.float32), Wd)[:, None], (Nw, Hd))
    b2b = jnp.broadcast_to(
        jnp.tile(b2.astype(jnp.float32), Wd)[:, None], (Nw, Hd))

    out = pl.pallas_call(
        functools.partial(_fused_kernel, Hu, Wu, Cin, Cout),
        out_shape=jax.ShapeDtypeStruct((N, Cout, Hd, Wd), bf16),
        grid=(N,),
        in_specs=[
            pl.BlockSpec((1, Cin, Hu, Wu), lambda n: (n, 0, 0, 0)),
            pl.BlockSpec((1, Cout, Hd, Wd), lambda n: (n, 0, 0, 0)),
            pl.BlockSpec((2, Nw, Ku), lambda n: (0, 0, 0)),
            pl.BlockSpec((3, Nw, Nw), lambda n: (0, 0, 0)),
            pl.BlockSpec((3, Nw, Nw), lambda n: (0, 0, 0)),
            pl.BlockSpec((3, Nw, Nw), lambda n: (0, 0, 0)),
            pl.BlockSpec((6, Hu, Hd), lambda n: (0, 0, 0)),
            pl.BlockSpec((2, Hd, Hd), lambda n: (0, 0, 0)),
            pl.BlockSpec((Wu, Wu), lambda n: (0, 0)),
            pl.BlockSpec((Wd, Wd), lambda n: (0, 0)),
            pl.BlockSpec((Wd, Wd), lambda n: (0, 0)),
            pl.BlockSpec((3, Nw, Hd), lambda n: (0, 0, 0)),
            pl.BlockSpec((Nw, Hd), lambda n: (0, 0)),
            pl.BlockSpec((Nw, Hd), lambda n: (0, 0)),
        ],
        out_specs=pl.BlockSpec((1, Cout, Hd, Wd), lambda n: (n, 0, 0, 0)),
        compiler_params=pltpu.CompilerParams(
            dimension_semantics=("parallel",),
            vmem_limit_bytes=64 * 1024 * 1024,
        ),
    )(from_up.astype(bf16), from_down.astype(bf16),
      mu, w1u, w1f, w2b, si, sh, eyu, eyd, eyf, btb, b1b, b2b)

    return out.astype(jnp.float32)


# speedup vs baseline: 1.2511x; 1.2511x over previous
import functools

import numpy as np
import jax
import jax.numpy as jnp
from jax.experimental import pallas as pl
from jax.experimental.pallas import tpu as pltpu


def _band_masks(Wd):
    m = np.zeros((3, Wd, Wd), np.float32)
    for j in range(Wd):
        for dx in range(3):
            i = j + dx - 1
            if 0 <= i < Wd:
                m[dx, j, i] = 1.0
    return m


def _up_masks(Wu):
    Wd = 2 * Wu
    m = np.zeros((2, Wd, Wu), np.float32)
    for w in range(Wu):
        for dj in range(2):
            m[dj, 2 * w + dj, w] = 1.0
    return m


def _block_sel(C, W):
    s = np.zeros((C * W, C), np.float32)
    for r in range(C * W):
        s[r, r // W] = 1.0
    return s


def _interleave_mats(H):
    s = np.zeros((2, H // 2, H), np.float32)
    for i in range(H // 2):
        s[0, i, 2 * i] = 1.0
        s[1, i, 2 * i + 1] = 1.0
    return s


def _builder_kernel(Cout, Cin, Wu,
                    w1p_ref, w2p_ref, wtp_ref, ro_ref, rcT_ref, ruT_ref,
                    mb_ref, mup_ref, w1u_ref, w1f_ref, w2b_ref, mu_ref):
    bf16 = jnp.bfloat16
    f32 = jnp.float32
    ro = ro_ref[...]
    rcT = rcT_ref[...]
    ruT = ruT_ref[...]

    def big(wsmall, rT):
        t = jnp.dot(ro, wsmall, preferred_element_type=f32)
        return jnp.dot(t, rT, preferred_element_type=f32)

    for k in range(3):
        au = af = az = None
        for dx in range(3):
            wkx = w1p_ref[3 * k + dx]
            m = mb_ref[dx]
            tu = big(wkx[:, :Cout], rcT) * m
            tf = big(wkx[:, Cout:], rcT) * m
            t2 = big(w2p_ref[3 * k + dx], rcT) * m
            au = tu if au is None else au + tu
            af = tf if af is None else af + tf
            az = t2 if az is None else az + t2
        w1u_ref[k] = au.astype(bf16)
        w1f_ref[k] = af.astype(bf16)
        w2b_ref[k] = az.astype(bf16)

    for p in range(2):
        acc = None
        for dj in range(2):
            t = big(wtp_ref[2 * p + dj], ruT) * mup_ref[dj]
            acc = t if acc is None else acc + t
        mu_ref[p] = acc.astype(bf16)


def _build_weight_mats(w1, w2, wt, Wu):
    Cout = w2.shape[0]
    Wd = 2 * Wu
    Nw = Cout * Wd
    Ku = wt.shape[0] * Wu
    f32 = jnp.float32
    w1p = jnp.transpose(w1, (2, 3, 0, 1)).reshape(9, Cout, 2 * Cout)
    w2p = jnp.transpose(w2, (2, 3, 0, 1)).reshape(9, Cout, Cout)
    wtp = jnp.transpose(wt, (2, 3, 1, 0)).reshape(4, Cout, wt.shape[0])
    ro = _block_sel(Cout, Wd)
    rcT = _block_sel(Cout, Wd).T
    ruT = _block_sel(wt.shape[0], Wu).T
    mb = np.tile(_band_masks(Wd), (1, Cout, Cout))
    mup = np.tile(_up_masks(Wu), (1, Cout, wt.shape[0]))

    return pl.pallas_call(
        functools.partial(_builder_kernel, Cout, wt.shape[0], Wu),
        out_shape=(
            jax.ShapeDtypeStruct((3, Nw, Nw), jnp.bfloat16),
            jax.ShapeDtypeStruct((3, Nw, Nw), jnp.bfloat16),
            jax.ShapeDtypeStruct((3, Nw, Nw), jnp.bfloat16),
            jax.ShapeDtypeStruct((2, Nw, Ku), jnp.bfloat16),
        ),
    )(w1p.astype(f32), w2p.astype(f32), wtp.astype(f32),
      jnp.asarray(ro), jnp.asarray(rcT), jnp.asarray(ruT),
      jnp.asarray(mb), jnp.asarray(mup))


def _dot(a, b):
    return jnp.dot(a, b, preferred_element_type=jnp.float32)


def _shl(x):
    return jnp.concatenate([jnp.zeros_like(x[:, :1]), x[:, :-1]], axis=1)


def _shr(x):
    return jnp.concatenate([x[:, 1:], jnp.zeros_like(x[:, :1])], axis=1)


def _fused_kernel(Hu, Wu, Cin, Cout,
                  fu_ref, fd_ref, mu_ref, w1u_ref, w1f_ref, w2_ref,
                  si_ref, btb_ref, b1b_ref, b2b_ref, o_ref):
    for b in range(fu_ref.shape[0]):
        _one_image(Hu, Wu, Cin, Cout, b,
                   fu_ref, fd_ref, mu_ref, w1u_ref, w1f_ref, w2_ref,
                   si_ref, btb_ref, b1b_ref, b2b_ref, o_ref)


def _one_image(Hu, Wu, Cin, Cout, b,
               fu_ref, fd_ref, mu_ref, w1u_ref, w1f_ref, w2_ref,
               si_ref, btb_ref, b1b_ref, b2b_ref, o_ref):
    bf16 = jnp.bfloat16
    Wd = 2 * Wu
    Hd = 2 * Hu
    Nw = Cout * Wd

    xfu = jnp.transpose(fu_ref[b], (0, 2, 1)).reshape(Cin * Wu, Hu)
    fdp = jnp.transpose(fd_ref[b], (0, 2, 1)).reshape(Nw, Hd)

    up_e = _dot(mu_ref[0], xfu).astype(bf16)
    up_o = _dot(mu_ref[1], xfu).astype(bf16)
    up = (_dot(up_e, si_ref[0]) + _dot(up_o, si_ref[1])
          + btb_ref[...]).astype(bf16)

    h1 = (_dot(w1u_ref[0], _shl(up)) + _dot(w1f_ref[0], _shl(fdp))
          + _dot(w1u_ref[1], up) + _dot(w1f_ref[1], fdp)
          + _dot(w1u_ref[2], _shr(up)) + _dot(w1f_ref[2], _shr(fdp)))
    h1 = jnp.maximum(h1 + b1b_ref[...], 0.0).astype(bf16)

    h2 = (_dot(w2_ref[0], _shl(h1)) + _dot(w2_ref[1], h1)
          + _dot(w2_ref[2], _shr(h1)))
    h2 = jnp.maximum(h2 + b2b_ref[...], 0.0).astype(bf16)

    o_ref[b] = jnp.transpose(h2.reshape(Cout, Wd, Hd), (0, 2, 1))


def kernel(from_down, from_up, wt, bt, w1, b1, w2, b2):
    N, Cout, Hd, Wd = from_down.shape
    _, Cin, Hu, Wu = from_up.shape
    bf16 = jnp.bfloat16
    Ku = Cin * Wu
    Nw = Cout * Wd

    w1u, w1f, w2b, mu = _build_weight_mats(w1, w2, wt, Wu)
    si = jnp.asarray(_interleave_mats(Hd), dtype=bf16)
    btb = jnp.broadcast_to(
        jnp.repeat(bt.astype(jnp.float32), Wd)[:, None], (Nw, Hd))
    b1b = jnp.broadcast_to(
        jnp.repeat(b1.astype(jnp.float32), Wd)[:, None], (Nw, Hd))
    b2b = jnp.broadcast_to(
        jnp.repeat(b2.astype(jnp.float32), Wd)[:, None], (Nw, Hd))

    ib = 2 if N % 2 == 0 else 1
    out = pl.pallas_call(
        functools.partial(_fused_kernel, Hu, Wu, Cin, Cout),
        out_shape=jax.ShapeDtypeStruct((N, Cout, Hd, Wd), bf16),
        grid=(N // ib,),
        in_specs=[
            pl.BlockSpec((ib, Cin, Hu, Wu), lambda n: (n, 0, 0, 0)),
            pl.BlockSpec((ib, Cout, Hd, Wd), lambda n: (n, 0, 0, 0)),
            pl.BlockSpec((2, Nw, Ku), lambda n: (0, 0, 0)),
            pl.BlockSpec((3, Nw, Nw), lambda n: (0, 0, 0)),
            pl.BlockSpec((3, Nw, Nw), lambda n: (0, 0, 0)),
            pl.BlockSpec((3, Nw, Nw), lambda n: (0, 0, 0)),
            pl.BlockSpec((2, Hu, Hd), lambda n: (0, 0, 0)),
            pl.BlockSpec((Nw, Hd), lambda n: (0, 0)),
            pl.BlockSpec((Nw, Hd), lambda n: (0, 0)),
            pl.BlockSpec((Nw, Hd), lambda n: (0, 0)),
        ],
        out_specs=pl.BlockSpec((ib, Cout, Hd, Wd), lambda n: (n, 0, 0, 0)),
        compiler_params=pltpu.CompilerParams(
            dimension_semantics=("parallel",),
            vmem_limit_bytes=64 * 1024 * 1024,
        ),
    )(from_up.astype(bf16), from_down.astype(bf16),
      mu, w1u, w1f, w2b, si, btb, b1b, b2b)

    return out.astype(jnp.float32)


# speedup vs baseline: 1.2576x; 1.0052x over previous
import functools

import numpy as np
import jax
import jax.numpy as jnp
from jax.experimental import pallas as pl
from jax.experimental.pallas import tpu as pltpu


def _band_masks(Wd):
    m = np.zeros((3, Wd, Wd), np.float32)
    for j in range(Wd):
        for dx in range(3):
            i = j + dx - 1
            if 0 <= i < Wd:
                m[dx, j, i] = 1.0
    return m


def _up_masks(Wu):
    Wd = 2 * Wu
    m = np.zeros((2, Wd, Wu), np.float32)
    for w in range(Wu):
        for dj in range(2):
            m[dj, 2 * w + dj, w] = 1.0
    return m


def _block_sel(C, W):
    s = np.zeros((C * W, C), np.float32)
    for r in range(C * W):
        s[r, r // W] = 1.0
    return s


def _interleave_mats(H):
    s = np.zeros((2, H // 2, H), np.float32)
    for i in range(H // 2):
        s[0, i, 2 * i] = 1.0
        s[1, i, 2 * i + 1] = 1.0
    return s


def _builder_kernel(Cout, Cin, Wu,
                    w1p_ref, w2p_ref, wtp_ref, ro_ref, rcT_ref, ruT_ref,
                    mb_ref, mup_ref, w1u_ref, w1f_ref, w2b_ref, mu_ref):
    bf16 = jnp.bfloat16
    f32 = jnp.float32
    ro = ro_ref[...]
    rcT = rcT_ref[...]
    ruT = ruT_ref[...]

    def big(wsmall, rT):
        t = jnp.dot(ro, wsmall, preferred_element_type=f32)
        return jnp.dot(t, rT, preferred_element_type=f32)

    for k in range(3):
        au = af = az = None
        for dx in range(3):
            wkx = w1p_ref[3 * k + dx]
            m = mb_ref[dx]
            tu = big(wkx[:, :Cout], rcT) * m
            tf = big(wkx[:, Cout:], rcT) * m
            t2 = big(w2p_ref[3 * k + dx], rcT) * m
            au = tu if au is None else au + tu
            af = tf if af is None else af + tf
            az = t2 if az is None else az + t2
        w1u_ref[k] = au.astype(bf16)
        w1f_ref[k] = af.astype(bf16)
        w2b_ref[k] = az.astype(bf16)

    for p in range(2):
        acc = None
        for dj in range(2):
            t = big(wtp_ref[2 * p + dj], ruT) * mup_ref[dj]
            acc = t if acc is None else acc + t
        mu_ref[p] = acc.astype(bf16)


def _build_weight_mats(w1, w2, wt, Wu):
    Cout = w2.shape[0]
    Wd = 2 * Wu
    Nw = Cout * Wd
    Ku = wt.shape[0] * Wu
    f32 = jnp.float32
    w1p = jnp.transpose(w1, (2, 3, 0, 1)).reshape(9, Cout, 2 * Cout)
    w2p = jnp.transpose(w2, (2, 3, 0, 1)).reshape(9, Cout, Cout)
    wtp = jnp.transpose(wt, (2, 3, 1, 0)).reshape(4, Cout, wt.shape[0])
    ro = _block_sel(Cout, Wd)
    rcT = _block_sel(Cout, Wd).T
    ruT = _block_sel(wt.shape[0], Wu).T
    mb = np.tile(_band_masks(Wd), (1, Cout, Cout))
    mup = np.tile(_up_masks(Wu), (1, Cout, wt.shape[0]))

    return pl.pallas_call(
        functools.partial(_builder_kernel, Cout, wt.shape[0], Wu),
        out_shape=(
            jax.ShapeDtypeStruct((3, Nw, Nw), jnp.bfloat16),
            jax.ShapeDtypeStruct((3, Nw, Nw), jnp.bfloat16),
            jax.ShapeDtypeStruct((3, Nw, Nw), jnp.bfloat16),
            jax.ShapeDtypeStruct((2, Nw, Ku), jnp.bfloat16),
        ),
    )(w1p.astype(f32), w2p.astype(f32), wtp.astype(f32),
      jnp.asarray(ro), jnp.asarray(rcT), jnp.asarray(ruT),
      jnp.asarray(mb), jnp.asarray(mup))


def _dot(a, b):
    return jnp.dot(a, b, preferred_element_type=jnp.float32)


def _shl(x):
    return jnp.concatenate([jnp.zeros_like(x[:, :1]), x[:, :-1]], axis=1)


def _shr(x):
    return jnp.concatenate([x[:, 1:], jnp.zeros_like(x[:, :1])], axis=1)


def _fused_kernel(Hu, Wu, Cin, Cout,
                  fu_ref, fd_ref, mu_ref, w1u_ref, w1f_ref, w2_ref,
                  si_ref, btb_ref, b1b_ref, b2b_ref, o_ref):
    for b in range(fu_ref.shape[0]):
        _one_image(Hu, Wu, Cin, Cout, b,
                   fu_ref, fd_ref, mu_ref, w1u_ref, w1f_ref, w2_ref,
                   si_ref, btb_ref, b1b_ref, b2b_ref, o_ref)


def _one_image(Hu, Wu, Cin, Cout, b,
               fu_ref, fd_ref, mu_ref, w1u_ref, w1f_ref, w2_ref,
               si_ref, btb_ref, b1b_ref, b2b_ref, o_ref):
    bf16 = jnp.bfloat16
    Wd = 2 * Wu
    Hd = 2 * Hu
    Nw = Cout * Wd

    xfu = jnp.transpose(fu_ref[b], (0, 2, 1)).reshape(Cin * Wu, Hu)
    fdp = jnp.transpose(fd_ref[b], (0, 2, 1)).reshape(Nw, Hd)

    up_e = _dot(mu_ref[0], xfu).astype(bf16)
    up_o = _dot(mu_ref[1], xfu).astype(bf16)
    up = (_dot(up_e, si_ref[0]) + _dot(up_o, si_ref[1])
          + btb_ref[...]).astype(bf16)

    h1 = (_dot(w1u_ref[0], _shl(up)) + _dot(w1f_ref[0], _shl(fdp))
          + _dot(w1u_ref[1], up) + _dot(w1f_ref[1], fdp)
          + _dot(w1u_ref[2], _shr(up)) + _dot(w1f_ref[2], _shr(fdp)))
    h1 = jnp.maximum(h1 + b1b_ref[...], 0.0).astype(bf16)

    h2 = (_dot(w2_ref[0], _shl(h1)) + _dot(w2_ref[1], h1)
          + _dot(w2_ref[2], _shr(h1)))
    h2 = jnp.maximum(h2 + b2b_ref[...], 0.0).astype(bf16)

    o_ref[b] = jnp.transpose(h2.reshape(Cout, Wd, Hd), (0, 2, 1))


def kernel(from_down, from_up, wt, bt, w1, b1, w2, b2):
    N, Cout, Hd, Wd = from_down.shape
    _, Cin, Hu, Wu = from_up.shape
    bf16 = jnp.bfloat16
    Ku = Cin * Wu
    Nw = Cout * Wd

    w1u, w1f, w2b, mu = _build_weight_mats(w1, w2, wt, Wu)
    si = jnp.asarray(_interleave_mats(Hd), dtype=bf16)
    btb = jnp.broadcast_to(
        jnp.repeat(bt.astype(jnp.float32), Wd)[:, None], (Nw, Hd))
    b1b = jnp.broadcast_to(
        jnp.repeat(b1.astype(jnp.float32), Wd)[:, None], (Nw, Hd))
    b2b = jnp.broadcast_to(
        jnp.repeat(b2.astype(jnp.float32), Wd)[:, None], (Nw, Hd))

    ib = 4 if N % 4 == 0 else (2 if N % 2 == 0 else 1)
    out = pl.pallas_call(
        functools.partial(_fused_kernel, Hu, Wu, Cin, Cout),
        out_shape=jax.ShapeDtypeStruct((N, Cout, Hd, Wd), bf16),
        grid=(N // ib,),
        in_specs=[
            pl.BlockSpec((ib, Cin, Hu, Wu), lambda n: (n, 0, 0, 0)),
            pl.BlockSpec((ib, Cout, Hd, Wd), lambda n: (n, 0, 0, 0)),
            pl.BlockSpec((2, Nw, Ku), lambda n: (0, 0, 0)),
            pl.BlockSpec((3, Nw, Nw), lambda n: (0, 0, 0)),
            pl.BlockSpec((3, Nw, Nw), lambda n: (0, 0, 0)),
            pl.BlockSpec((3, Nw, Nw), lambda n: (0, 0, 0)),
            pl.BlockSpec((2, Hu, Hd), lambda n: (0, 0, 0)),
            pl.BlockSpec((Nw, Hd), lambda n: (0, 0)),
            pl.BlockSpec((Nw, Hd), lambda n: (0, 0)),
            pl.BlockSpec((Nw, Hd), lambda n: (0, 0)),
        ],
        out_specs=pl.BlockSpec((ib, Cout, Hd, Wd), lambda n: (n, 0, 0, 0)),
        compiler_params=pltpu.CompilerParams(
            dimension_semantics=("parallel",),
            vmem_limit_bytes=64 * 1024 * 1024,
        ),
    )(from_up.astype(bf16), from_down.astype(bf16),
      mu, w1u, w1f, w2b, si, btb, b1b, b2b)

    return out.astype(jnp.float32)
